# baseline (speedup 1.0000x reference)
import functools

import jax
import jax.numpy as jnp
from jax import lax
from jax.experimental import pallas as pl
from jax.experimental.pallas import tpu as pltpu
from jax.experimental.pallas import tpu_sc as plsc

EMB = 64
NC = 2
NS = 16
NW = NC * NS
VOC = 1000000
TCOLS = 7811
VTAIL = VOC - TCOLS * 128
SROWS = VOC * EMB // 128
TROWS = VTAIL * EMB // 128

BATCH = 16384
HIST = 50
CB = 256
UNITS = HIST * (BATCH // CB) // NW


def _mesh():
    return plsc.VectorSubcoreMesh(
        core_axis_name="c", subcore_axis_name="s", num_cores=NC, num_subcores=NS
    )


def _wid():
    return lax.axis_index("s") * NC + lax.axis_index("c")


def _splat(v):
    return jnp.full((16,), v, dtype=jnp.int32)


@functools.partial(
    pl.kernel,
    out_type=jax.ShapeDtypeStruct((SROWS, 128), jnp.float32),
    mesh=_mesh(),
    scratch_types=[
        pltpu.VMEM((64, 128), jnp.float32),
        pltpu.VMEM((64, 128), jnp.float32),
        pltpu.VMEM((64, 128), jnp.float32),
        pltpu.VMEM((64, 128), jnp.float32),
        pltpu.SemaphoreType.DMA,
        pltpu.SemaphoreType.DMA,
        pltpu.SemaphoreType.DMA,
        pltpu.SemaphoreType.DMA,
    ],
    compiler_params=pltpu.CompilerParams(
        use_tc_tiling_on_sc=True, needs_layout_passes=False
    ),
)
def _stage_table(wt_hbm, tail_hbm, st_hbm, ib0, ib1, ob0, ob1, i0, i1, o0, o1):
    wid = _wid()
    ib = (ib0, ib1)
    ob = (ob0, ob1)
    isem = (i0, i1)
    osem = (o0, o1)
    iota = lax.iota(jnp.int32, 16)
    rows4 = [iota + (j * 16) for j in range(4)]

    def tc_of(k):
        return wid + 32 * k

    def i_start(k, p):
        pltpu.async_copy(
            wt_hbm.at[:, pl.ds(tc_of(k) * 128, 128)], ib[p], isem[p]
        )

    def i_wait(p):
        pltpu.make_async_copy(
            wt_hbm.at[:, pl.ds(0, 128)], ib[p], isem[p]
        ).wait()

    def o_start(k, p):
        pltpu.async_copy(
            ob[p], st_hbm.at[pl.ds(tc_of(k) * 64, 64)], osem[p]
        )

    def o_wait(p):
        pltpu.make_async_copy(
            ob[p], st_hbm.at[pl.ds(0, 64)], osem[p]
        ).wait()

    def transpose(p):
        src, dst = ib[p], ob[p]

        @pl.loop(0, 64)
        def _(rr):
            for cc0 in range(8):
                cols = _splat(2 * rr + (1 if cc0 >= 4 else 0))
                vals = plsc.load_gather(src, [rows4[cc0 % 4], cols])
                dst[rr, pl.ds(cc0 * 16, 16)] = vals

    i_start(0, 0)

    @pl.loop(0, 122)
    def _(kh):
        k0 = kh * 2
        i_start(k0 + 1, 1)
        i_wait(0)

        @pl.when(k0 >= 2)
        def _():
            o_wait(0)

        transpose(0)
        o_start(k0, 0)

        @pl.when(jnp.logical_or(k0 != 242, wid < 3))
        def _():
            i_start(k0 + 2, 0)

        i_wait(1)

        @pl.when(k0 > 0)
        def _():
            o_wait(1)

        transpose(1)
        o_start(k0 + 1, 1)

    @pl.when(wid < 3)
    def _():
        i_wait(0)
        o_wait(0)
        transpose(0)
        o_start(244, 0)

    o_wait(1)
    o_wait(0)

    @pl.when(wid == 31)
    def _():
        pltpu.sync_copy(tail_hbm.at[pl.ds(0, 64)], ib0)
        pltpu.sync_copy(ib0, st_hbm.at[pl.ds(TCOLS * 64, 64)])
        pltpu.sync_copy(tail_hbm.at[pl.ds(64, 32)], ib1.at[pl.ds(0, 32)])
        pltpu.sync_copy(
            ib1.at[pl.ds(0, 32)], st_hbm.at[pl.ds(TCOLS * 64 + 64, 32)]
        )


@functools.partial(
    pl.kernel,
    out_type=jax.ShapeDtypeStruct((HIST, 8, 128, 8, 128), jnp.float32),
    mesh=_mesh(),
    scratch_types=[
        pltpu.VMEM((CB,), jnp.int32),
        pltpu.VMEM((CB,), jnp.int32),
        pltpu.VMEM((CB, EMB), jnp.float32),
        pltpu.VMEM((CB, EMB), jnp.float32),
        pltpu.VMEM((8, 2, 8, 128), jnp.float32),
        pltpu.VMEM((8, 2, 8, 128), jnp.float32),
        pltpu.SemaphoreType.DMA,
        pltpu.SemaphoreType.DMA,
        pltpu.SemaphoreType.DMA,
        pltpu.SemaphoreType.DMA,
        pltpu.SemaphoreType.DMA,
        pltpu.SemaphoreType.DMA,
    ],
    compiler_params=pltpu.CompilerParams(
        use_tc_tiling_on_sc=False, needs_layout_passes=False
    ),
)
def _gather_t(xt_hbm, st_hbm, out_hbm, ibuf0, ibuf1, g0, g1, obuf0, obuf1,
              is0, is1, gs0, gs1, os0, os1):
    wid = _wid()
    ibuf = (ibuf0, ibuf1)
    gbuf = (g0, g1)
    obuf = (obuf0, obuf1)
    isem = (is0, is1)
    gsem = (gs0, gs1)
    osem = (os0, os1)
    iota = lax.iota(jnp.int32, 16)
    rows16 = [iota + (j * 16) for j in range(16)]

    def h_of(u):
        return u // 2

    def b0_of(u):
        return wid * 512 + (u % 2) * CB

    def c0_of(u):
        return wid * 4 + (u % 2) * 2

    def x_start(u, p):
        pltpu.async_copy(
            xt_hbm.at[h_of(u), pl.ds(b0_of(u), CB)], ibuf[p], isem[p]
        )

    def x_wait(p):
        pltpu.make_async_copy(
            xt_hbm.at[0, pl.ds(0, CB)], ibuf[p], isem[p]
        ).wait()

    def g_start(p):
        pltpu.async_copy(st_hbm.at[ibuf[p]], gbuf[p], gsem[p])

    def g_wait(p):
        pltpu.make_async_copy(st_hbm.at[ibuf[p]], gbuf[p], gsem[p]).wait()

    def o_start(u, p):
        for r in range(8):
            pltpu.async_copy(
                obuf[p].at[r],
                out_hbm.at[h_of(u), r, pl.ds(c0_of(u), 2)],
                osem[p],
            )

    def o_wait(p):
        for _ in range(8):
            pltpu.make_async_copy(
                obuf[p].at[0], out_hbm.at[0, 0, pl.ds(0, 2)], osem[p]
            ).wait()

    def transpose(p):
        src, dst = gbuf[p], obuf[p]

        @pl.loop(0, EMB)
        def _(f):
            cols = _splat(f)
            r = f // 8
            fr = f % 8
            for j in range(16):
                vals = plsc.load_gather(src, [rows16[j], cols])
                dst[r, j // 8, fr, pl.ds((j % 8) * 16, 16)] = vals

    x_start(0, 0)
    x_start(1, 1)
    x_wait(0)
    g_start(0)

    @pl.loop(0, UNITS // 2)
    def _(uh):
        u0 = uh * 2
        x_wait(1)
        g_start(1)
        g_wait(0)

        @pl.when(u0 != UNITS - 2)
        def _():
            x_start(u0 + 2, 0)

        @pl.when(u0 >= 2)
        def _():
            o_wait(0)

        transpose(0)
        o_start(u0, 0)

        @pl.when(u0 != UNITS - 2)
        def _():
            x_wait(0)
            g_start(0)

        g_wait(1)

        @pl.when(u0 != UNITS - 2)
        def _():
            x_start(u0 + 3, 1)

        @pl.when(u0 > 0)
        def _():
            o_wait(1)

        transpose(1)
        o_start(u0 + 1, 1)

    o_wait(0)
    o_wait(1)


def kernel(x, emb_weight):
    xt = x.T.astype(jnp.int32)
    tail = emb_weight[TCOLS * 128 :].reshape(TROWS, 128)
    staged = _stage_table(emb_weight.T, tail)
    staged_lin = staged.reshape(VOC, EMB)
    out5 = _gather_t(xt, staged_lin)
    return (
        out5.transpose(0, 1, 3, 2, 4)
        .reshape(HIST, EMB, BATCH)
        .transpose(2, 0, 1)
    )


# speedup vs baseline: 1.4896x; 1.4896x over previous
import functools

import jax
import jax.numpy as jnp
from jax import lax
from jax.experimental import pallas as pl
from jax.experimental.pallas import tpu as pltpu
from jax.experimental.pallas import tpu_sc as plsc

EMB = 64
NC = 2
NS = 16
NW = NC * NS
VOC = 1000000
TCOLS = 7811
VTAIL = VOC - TCOLS * 128
SROWS = VOC * EMB // 128
TROWS = VTAIL * EMB // 128

BATCH = 16384
HIST = 50
CB = 256
UNITS = HIST * (BATCH // CB) // NW


def _mesh():
    return plsc.VectorSubcoreMesh(
        core_axis_name="c", subcore_axis_name="s", num_cores=NC, num_subcores=NS
    )


def _wid():
    return lax.axis_index("s") * NC + lax.axis_index("c")


def _splat(v):
    return jnp.full((16,), v, dtype=jnp.int32)


@functools.partial(
    pl.kernel,
    out_type=jax.ShapeDtypeStruct((SROWS, 128), jnp.float32),
    mesh=_mesh(),
    scratch_types=[
        pltpu.VMEM((64, 128), jnp.float32),
        pltpu.VMEM((64, 128), jnp.float32),
        pltpu.VMEM((64, 128), jnp.float32),
        pltpu.VMEM((64, 128), jnp.float32),
        pltpu.SemaphoreType.DMA,
        pltpu.SemaphoreType.DMA,
        pltpu.SemaphoreType.DMA,
        pltpu.SemaphoreType.DMA,
    ],
    compiler_params=pltpu.CompilerParams(
        use_tc_tiling_on_sc=True, needs_layout_passes=False
    ),
)
def _stage_table(wt_hbm, tail_hbm, st_hbm, ib0, ib1, ob0, ob1, i0, i1, o0, o1):
    wid = _wid()
    ib = (ib0, ib1)
    ob = (ob0, ob1)
    isem = (i0, i1)
    osem = (o0, o1)
    iota = lax.iota(jnp.int32, 16)
    rows4 = [iota + (j * 16) for j in range(4)]

    def tc_of(k):
        return wid + 32 * k

    def i_start(k, p):
        pltpu.async_copy(
            wt_hbm.at[:, pl.ds(tc_of(k) * 128, 128)], ib[p], isem[p]
        )

    def i_wait(p):
        pltpu.make_async_copy(
            wt_hbm.at[:, pl.ds(0, 128)], ib[p], isem[p]
        ).wait()

    def o_start(k, p):
        pltpu.async_copy(
            ob[p], st_hbm.at[pl.ds(tc_of(k) * 64, 64)], osem[p]
        )

    def o_wait(p):
        pltpu.make_async_copy(
            ob[p], st_hbm.at[pl.ds(0, 64)], osem[p]
        ).wait()

    def transpose(p):
        src, dst = ib[p], ob[p]

        @plsc.parallel_loop(0, 64, unroll=4)
        def _(rr):
            vals = [
                plsc.load_gather(
                    src,
                    [rows4[cc0 % 4], _splat(2 * rr + (1 if cc0 >= 4 else 0))],
                )
                for cc0 in range(8)
            ]
            for cc0 in range(8):
                dst[rr, pl.ds(cc0 * 16, 16)] = vals[cc0]

    i_start(0, 0)

    @pl.loop(0, 122)
    def _(kh):
        k0 = kh * 2
        i_start(k0 + 1, 1)
        i_wait(0)

        @pl.when(k0 >= 2)
        def _():
            o_wait(0)

        transpose(0)
        o_start(k0, 0)

        @pl.when(jnp.logical_or(k0 != 242, wid < 3))
        def _():
            i_start(k0 + 2, 0)

        i_wait(1)

        @pl.when(k0 > 0)
        def _():
            o_wait(1)

        transpose(1)
        o_start(k0 + 1, 1)

    @pl.when(wid < 3)
    def _():
        i_wait(0)
        o_wait(0)
        transpose(0)
        o_start(244, 0)

    o_wait(1)
    o_wait(0)

    @pl.when(wid == 31)
    def _():
        pltpu.sync_copy(tail_hbm.at[pl.ds(0, 64)], ib0)
        pltpu.sync_copy(ib0, st_hbm.at[pl.ds(TCOLS * 64, 64)])
        pltpu.sync_copy(tail_hbm.at[pl.ds(64, 32)], ib1.at[pl.ds(0, 32)])
        pltpu.sync_copy(
            ib1.at[pl.ds(0, 32)], st_hbm.at[pl.ds(TCOLS * 64 + 64, 32)]
        )


@functools.partial(
    pl.kernel,
    out_type=jax.ShapeDtypeStruct((HIST, 8, 128, 8, 128), jnp.float32),
    mesh=_mesh(),
    scratch_types=[
        pltpu.VMEM((CB,), jnp.int32),
        pltpu.VMEM((CB,), jnp.int32),
        pltpu.VMEM((CB, EMB), jnp.float32),
        pltpu.VMEM((CB, EMB), jnp.float32),
        pltpu.VMEM((8, 2, 8, 128), jnp.float32),
        pltpu.VMEM((8, 2, 8, 128), jnp.float32),
        pltpu.SemaphoreType.DMA,
        pltpu.SemaphoreType.DMA,
        pltpu.SemaphoreType.DMA,
        pltpu.SemaphoreType.DMA,
        pltpu.SemaphoreType.DMA,
        pltpu.SemaphoreType.DMA,
    ],
    compiler_params=pltpu.CompilerParams(
        use_tc_tiling_on_sc=False, needs_layout_passes=False
    ),
)
def _gather_t(xt_hbm, st_hbm, out_hbm, ibuf0, ibuf1, g0, g1, obuf0, obuf1,
              is0, is1, gs0, gs1, os0, os1):
    wid = _wid()
    ibuf = (ibuf0, ibuf1)
    gbuf = (g0, g1)
    obuf = (obuf0, obuf1)
    isem = (is0, is1)
    gsem = (gs0, gs1)
    osem = (os0, os1)
    iota = lax.iota(jnp.int32, 16)
    rows16 = [iota + (j * 16) for j in range(16)]

    def h_of(u):
        return u // 2

    def b0_of(u):
        return wid * 512 + (u % 2) * CB

    def c0_of(u):
        return wid * 4 + (u % 2) * 2

    def x_start(u, p):
        pltpu.async_copy(
            xt_hbm.at[h_of(u), pl.ds(b0_of(u), CB)], ibuf[p], isem[p]
        )

    def x_wait(p):
        pltpu.make_async_copy(
            xt_hbm.at[0, pl.ds(0, CB)], ibuf[p], isem[p]
        ).wait()

    def g_start(p):
        pltpu.async_copy(st_hbm.at[ibuf[p]], gbuf[p], gsem[p])

    def g_wait(p):
        pltpu.make_async_copy(st_hbm.at[ibuf[p]], gbuf[p], gsem[p]).wait()

    def o_start(u, p):
        for r in range(8):
            pltpu.async_copy(
                obuf[p].at[r],
                out_hbm.at[h_of(u), r, pl.ds(c0_of(u), 2)],
                osem[p],
            )

    def o_wait(p):
        for _ in range(8):
            pltpu.make_async_copy(
                obuf[p].at[0], out_hbm.at[0, 0, pl.ds(0, 2)], osem[p]
            ).wait()

    def transpose(p):
        src, dst = gbuf[p], obuf[p]

        @plsc.parallel_loop(0, EMB, unroll=4)
        def _(f):
            cols = _splat(f)
            r = f // 8
            fr = f % 8
            vals = [plsc.load_gather(src, [rows16[j], cols]) for j in range(16)]
            for j in range(16):
                dst[r, j // 8, fr, pl.ds((j % 8) * 16, 16)] = vals[j]

    x_start(0, 0)
    x_start(1, 1)
    x_wait(0)
    g_start(0)

    @pl.loop(0, UNITS // 2)
    def _(uh):
        u0 = uh * 2
        x_wait(1)
        g_start(1)
        g_wait(0)

        @pl.when(u0 != UNITS - 2)
        def _():
            x_start(u0 + 2, 0)

        @pl.when(u0 >= 2)
        def _():
            o_wait(0)

        transpose(0)
        o_start(u0, 0)

        @pl.when(u0 != UNITS - 2)
        def _():
            x_wait(0)
            g_start(0)

        g_wait(1)

        @pl.when(u0 != UNITS - 2)
        def _():
            x_start(u0 + 3, 1)

        @pl.when(u0 > 0)
        def _():
            o_wait(1)

        transpose(1)
        o_start(u0 + 1, 1)

    o_wait(0)
    o_wait(1)


def kernel(x, emb_weight):
    xt = x.T.astype(jnp.int32)
    tail = emb_weight[TCOLS * 128 :].reshape(TROWS, 128)
    staged = _stage_table(emb_weight.T, tail)
    staged_lin = staged.reshape(VOC, EMB)
    out5 = _gather_t(xt, staged_lin)
    return (
        out5.transpose(0, 1, 3, 2, 4)
        .reshape(HIST, EMB, BATCH)
        .transpose(2, 0, 1)
    )
